# Initial kernel scaffold; baseline (speedup 1.0000x reference)
import jax
import jax.numpy as jnp
from jax.experimental import pallas as pl


def kernel(x, t, lap_down, lap_up, tW1, tb1, tW2, tb2, snn_w0, snn_w1, mapW, mapb, outW1, outb1, outW2, outb2):
    raise NotImplementedError("write your pallas kernel here")


# speedup vs baseline: 1.0268x; 1.0268x over previous
import math

import jax
import jax.numpy as jnp
from jax.experimental import pallas as pl
from jax.experimental.pallas import tpu as pltpu

_D = 4096
_B = 8
_HID = 64
_TDIM = 128
_BN = 512
_NB = _D // _BN
_F32 = jnp.float32


def _prep_body(t_ref, freqs_ref, tw1_ref, tb1_ref, tw2_ref, tb2_ref,
               w0_ref, w1_ref, tout_ref, coef_ref):
    args = t_ref[...] * freqs_ref[...]
    cosr = jnp.cos(args)
    sinr = jnp.sin(args)
    h = jnp.dot(cosr, tw1_ref[0:_TDIM // 2, :], preferred_element_type=_F32)
    h = h + jnp.dot(sinr, tw1_ref[_TDIM // 2:_TDIM, :], preferred_element_type=_F32)
    h = h + tb1_ref[...]
    h = h * jax.lax.logistic(h)
    tout_ref[...] = jnp.dot(h, tw2_ref[...], preferred_element_type=_F32) + tb2_ref[...]

    a = w0_ref[:, 0:1]
    b = w0_ref[:, 1:2]
    c = w0_ref[:, 2:3]
    p = w1_ref[:, 0:1]
    q = w1_ref[:, 1:2]
    r = w1_ref[:, 2:3]

    def s(u, v):
        return jnp.sum(u * v, axis=0, keepdims=True)

    c1 = s(a, p)
    c2 = s(b, p) + s(a, q)
    c3 = s(c, p) + s(a, r)
    c4 = s(b, q)
    c5 = s(c, q)
    c6 = s(b, r)
    c7 = s(c, r)
    coef_ref[...] = jnp.concatenate([c1, c2, c3, c4, c5, c6, c7, c1], axis=1)


def _pass1_body(ld_ref, lu_ref, xt_ref, coef_ref, pt_ref, qt_ref, rt_ref):
    ut = jnp.dot(ld_ref[...], xt_ref[...], preferred_element_type=_F32)
    vt = jnp.dot(lu_ref[...], xt_ref[...], preferred_element_type=_F32)
    c = coef_ref[...]
    rt_ref[...] = c[0:1, 1:2] * ut + c[0:1, 2:3] * vt
    pt_ref[...] = c[0:1, 3:4] * ut + c[0:1, 4:5] * vt
    qt_ref[...] = c[0:1, 5:6] * ut + c[0:1, 6:7] * vt


def _pass2_body(ld_ref, lu_ref, pt_ref, qt_ref, xt_ref, rt_ref, coef_ref, xo_ref):
    st = jnp.dot(ld_ref[...], pt_ref[...], preferred_element_type=_F32)
    st = st + jnp.dot(lu_ref[...], qt_ref[...], preferred_element_type=_F32)
    xo_ref[...] = coef_ref[0:1, 0:1] * xt_ref[...] + rt_ref[...] + st


def _head_body(xo_ref, mapwt_ref, mapb_ref, tout_ref, ow1_ref, ob1_ref,
               ow2_ref, ob2_ref, out_ref):
    x2t = jnp.dot(mapwt_ref[...], xo_ref[...], preferred_element_type=_F32)
    h = jnp.transpose(x2t) + mapb_ref[...] + tout_ref[...]
    h = jnp.dot(h, ow1_ref[...], preferred_element_type=_F32) + ob1_ref[...]
    h = h * jax.lax.logistic(h)
    out_ref[...] = jnp.dot(h, ow2_ref[...], preferred_element_type=_F32) + ob2_ref[...]


def kernel(x, t, lap_down, lap_up, tW1, tb1, tW2, tb2, snn_w0, snn_w1,
           mapW, mapb, outW1, outb1, outW2, outb2):
    xt = x.T
    t2 = t.reshape(_B, 1)
    half = _TDIM // 2
    freqs = jnp.exp(
        -math.log(10000.0) * jnp.arange(0, half, dtype=_F32) / half
    ).reshape(1, half)
    w0r = snn_w0[0]
    w1r = snn_w1[:, 0, :]
    tb1r = tb1.reshape(1, _HID)
    tb2r = tb2.reshape(1, _HID)
    mapbr = mapb.reshape(1, _HID)
    ob1r = outb1.reshape(1, _HID)
    ob2r = outb2.reshape(1, _D)
    mapwt = mapW.T

    tout, coefs = pl.pallas_call(
        _prep_body,
        out_shape=(
            jax.ShapeDtypeStruct((_B, _HID), _F32),
            jax.ShapeDtypeStruct((1, 8), _F32),
        ),
    )(t2, freqs, tW1, tb1r, tW2, tb2r, w0r, w1r)

    row_blk = pl.BlockSpec((_BN, _D), lambda i: (i, 0))
    vec_blk = pl.BlockSpec((_BN, _B), lambda i: (i, 0))
    full_vec = pl.BlockSpec((_D, _B), lambda i: (0, 0))
    coef_spec = pl.BlockSpec((1, 8), lambda i: (0, 0))

    pt, qt, rt = pl.pallas_call(
        _pass1_body,
        grid=(_NB,),
        in_specs=[row_blk, row_blk, full_vec, coef_spec],
        out_specs=(vec_blk, vec_blk, vec_blk),
        out_shape=(jax.ShapeDtypeStruct((_D, _B), _F32),) * 3,
    )(lap_down, lap_up, xt, coefs)

    xo = pl.pallas_call(
        _pass2_body,
        grid=(_NB,),
        in_specs=[row_blk, row_blk, full_vec, full_vec, vec_blk, vec_blk, coef_spec],
        out_specs=vec_blk,
        out_shape=jax.ShapeDtypeStruct((_D, _B), _F32),
    )(lap_down, lap_up, pt, qt, xt, rt, coefs)

    out = pl.pallas_call(
        _head_body,
        out_shape=jax.ShapeDtypeStruct((_B, _D), _F32),
    )(xo, mapwt, mapbr, tout, outW1, ob1r, outW2, ob2r)
    return out
